# Initial kernel scaffold; baseline (speedup 1.0000x reference)
import jax
import jax.numpy as jnp
from jax.experimental import pallas as pl


def kernel(x, edge_index, W1, b1, g1, be1, W2, b2, g2, be2, g3, be3):
    raise NotImplementedError("write your pallas kernel here")


# speedup vs baseline: 3.2228x; 3.2228x over previous
import functools

import jax
import jax.numpy as jnp
from jax import lax
from jax.experimental import pallas as pl
from jax.experimental.pallas import tpu as pltpu
from jax.experimental.pallas import tpu_sc as plsc

N = 10000
E = 320000
D = 128
DH = D // 2
BN_EPS = 1e-5

NC = 2
NS = 16
EPT = E // NS
CHUNK = 80
NCHUNK = EPT // CHUNK
NPAD = 10240
RPT = NPAD // NS


def _sc_agg_kernel(xa_hbm, xb_hbm, src_hbm, dst_hbm, out_hbm,
                   sidx, didx, rows, zbuf, agg_sh, sem):
    c = lax.axis_index("c")
    s = lax.axis_index("s")

    zv = jnp.zeros((16,), jnp.float32)

    def zbody(k, _):
        r = k // (DH // 16)
        cc = k % (DH // 16)
        zbuf[r, pl.ds(cc * 16, 16)] = zv
        return 0

    lax.fori_loop(0, RPT * (DH // 16), zbody, 0)
    pltpu.sync_copy(zbuf, agg_sh.at[pl.ds(s * RPT, RPT)])
    plsc.subcore_barrier()

    def make_body(x_hbm):
        def body(i, _):
            base = s * EPT + i * CHUNK
            pltpu.sync_copy(src_hbm.at[pl.ds(base, CHUNK)], sidx)
            pltpu.sync_copy(dst_hbm.at[pl.ds(base, CHUNK)], didx)
            pltpu.async_copy(x_hbm.at[sidx], rows, sem).wait()
            pltpu.sync_copy(rows, agg_sh.at[didx], add=True)
            return 0
        return body

    @pl.when(c == 0)
    def _():
        lax.fori_loop(0, NCHUNK, make_body(xa_hbm), 0)

    @pl.when(c == 1)
    def _():
        lax.fori_loop(0, NCHUNK, make_body(xb_hbm), 0)

    plsc.subcore_barrier()
    pl.delay(30000)
    plsc.subcore_barrier()

    pltpu.sync_copy(agg_sh.at[pl.ds(s * RPT, RPT)], zbuf)
    pltpu.sync_copy(zbuf, out_hbm.at[c, pl.ds(s * RPT, RPT)])


_sc_agg = functools.partial(
    pl.kernel,
    out_type=jax.ShapeDtypeStruct((NC, NPAD, DH), jnp.float32),
    mesh=plsc.VectorSubcoreMesh(core_axis_name="c", subcore_axis_name="s"),
    compiler_params=pltpu.CompilerParams(use_tc_tiling_on_sc=False),
    scratch_types=[
        pltpu.VMEM((CHUNK,), jnp.int32),
        pltpu.VMEM((CHUNK,), jnp.int32),
        pltpu.VMEM((CHUNK, DH), jnp.float32),
        pltpu.VMEM((RPT, DH), jnp.float32),
        pltpu.VMEM_SHARED((NPAD, DH), jnp.float32),
        pltpu.SemaphoreType.DMA,
    ],
)(_sc_agg_kernel)


def _bn(h, gamma, beta):
    mean = jnp.mean(h, axis=0, keepdims=True)
    var = jnp.mean((h - mean) ** 2, axis=0, keepdims=True)
    return (h - mean) * jax.lax.rsqrt(var + BN_EPS) * gamma + beta


def _mlp_kernel(x_ref, p0_ref, p1_ref, w1t_ref, b1_ref, g1_ref, be1_ref,
                w2t_ref, b2_ref, g2_ref, be2_ref, g3_ref, be3_ref, out_ref):
    agg = jnp.concatenate([p0_ref[...], p1_ref[...]], axis=1)
    h = x_ref[...] + agg
    h = jnp.dot(h, w1t_ref[...], preferred_element_type=jnp.float32)
    h = _bn(h + b1_ref[...], g1_ref[...], be1_ref[...])
    h = jnp.maximum(h, 0.0)
    h = jnp.dot(h, w2t_ref[...], preferred_element_type=jnp.float32)
    h = _bn(h + b2_ref[...], g2_ref[...], be2_ref[...])
    h = jnp.maximum(h, 0.0)
    out_ref[...] = _bn(h, g3_ref[...], be3_ref[...])


def kernel(x, edge_index, W1, b1, g1, be1, W2, b2, g2, be2, g3, be3):
    src = edge_index[0]
    dst = edge_index[1]
    xa = x[:, :DH]
    xb = x[:, DH:]
    parts = _sc_agg(xa, xb, src, dst)
    row = lambda v: v.reshape(1, -1)
    return pl.pallas_call(
        _mlp_kernel,
        out_shape=jax.ShapeDtypeStruct((N, D), jnp.float32),
    )(x, parts[0, :N], parts[1, :N], W1.T, row(b1), row(g1), row(be1),
      W2.T, row(b2), row(g2), row(be2), row(g3), row(be3))


# speedup vs baseline: 9.9457x; 3.0860x over previous
import functools

import jax
import jax.numpy as jnp
from jax import lax
from jax.experimental import pallas as pl
from jax.experimental.pallas import tpu as pltpu
from jax.experimental.pallas import tpu_sc as plsc

N = 10000
E = 320000
D = 128
DH = D // 2
BN_EPS = 1e-5

NC = 2
NS = 16
EPT = E // NS
CHUNK = 80
NCHUNK = EPT // CHUNK
NPAD = 10240
RPT = NPAD // NS
PDEPTH = 4
NBUF = 2 * PDEPTH
ZROWS = 80


def _sc_agg_kernel(xa_hbm, xb_hbm, src_hbm, dst_hbm, out_hbm,
                   sidx, didx, rows, zbuf, agg_sh, gsem, ssem):
    c = lax.axis_index("c")
    s = lax.axis_index("s")

    zv = jnp.zeros((16,), jnp.float32)

    def zbody(k, _):
        r = k // (DH // 16)
        cc = k % (DH // 16)
        zbuf[r, pl.ds(cc * 16, 16)] = zv
        return 0

    lax.fori_loop(0, ZROWS * (DH // 16), zbody, 0)

    def zcopy(z, _):
        pltpu.sync_copy(zbuf, agg_sh.at[pl.ds(s * RPT + z * ZROWS, ZROWS)])
        return 0

    lax.fori_loop(0, RPT // ZROWS, zcopy, 0)

    pltpu.sync_copy(src_hbm.at[s], sidx)
    pltpu.sync_copy(dst_hbm.at[s], didx)
    plsc.subcore_barrier()

    def make_loop(x_hbm):
        def body(i, _):
            @pl.when(i < NCHUNK)
            def _():
                k = lax.rem(i, NBUF)

                @pl.when(i >= NBUF)
                def _():
                    pltpu.make_async_copy(
                        rows.at[k], agg_sh.at[didx.at[0]], ssem.at[k]
                    ).wait()

                pltpu.make_async_copy(
                    x_hbm.at[sidx.at[i]], rows.at[k], gsem.at[k]
                ).start()

            j = i - PDEPTH

            @pl.when(j >= 0)
            def _():
                kj = lax.rem(j, NBUF)
                pltpu.make_async_copy(
                    x_hbm.at[sidx.at[j]], rows.at[kj], gsem.at[kj]
                ).wait()
                pltpu.make_async_copy(
                    rows.at[kj], agg_sh.at[didx.at[j]], ssem.at[kj]
                ).start(add=True)

            return 0

        lax.fori_loop(0, NCHUNK + PDEPTH, body, 0)

        def drain(k, _):
            pltpu.make_async_copy(
                rows.at[k], agg_sh.at[didx.at[0]], ssem.at[k]
            ).wait()
            return 0

        lax.fori_loop(0, NBUF, drain, 0)

    @pl.when(c == 0)
    def _():
        make_loop(xa_hbm)

    @pl.when(c == 1)
    def _():
        make_loop(xb_hbm)

    plsc.subcore_barrier()
    pl.delay(20000)
    plsc.subcore_barrier()

    pltpu.sync_copy(agg_sh.at[pl.ds(s * RPT, RPT)],
                    out_hbm.at[c, pl.ds(s * RPT, RPT)])


_sc_agg = functools.partial(
    pl.kernel,
    out_type=jax.ShapeDtypeStruct((NC, NPAD, DH), jnp.float32),
    mesh=plsc.VectorSubcoreMesh(core_axis_name="c", subcore_axis_name="s"),
    compiler_params=pltpu.CompilerParams(use_tc_tiling_on_sc=False),
    scratch_types=[
        pltpu.VMEM((NCHUNK, CHUNK), jnp.int32),
        pltpu.VMEM((NCHUNK, CHUNK), jnp.int32),
        pltpu.VMEM((NBUF, CHUNK, DH), jnp.float32),
        pltpu.VMEM((ZROWS, DH), jnp.float32),
        pltpu.VMEM_SHARED((NPAD, DH), jnp.float32),
        pltpu.SemaphoreType.DMA((NBUF,)),
        pltpu.SemaphoreType.DMA((NBUF,)),
    ],
)(_sc_agg_kernel)


def _bn(h, gamma, beta):
    mean = jnp.mean(h, axis=0, keepdims=True)
    var = jnp.mean((h - mean) ** 2, axis=0, keepdims=True)
    return (h - mean) * jax.lax.rsqrt(var + BN_EPS) * gamma + beta


def _mlp_kernel(x_ref, p0_ref, p1_ref, w1t_ref, b1_ref, g1_ref, be1_ref,
                w2t_ref, b2_ref, g2_ref, be2_ref, g3_ref, be3_ref, out_ref):
    agg = jnp.concatenate([p0_ref[...], p1_ref[...]], axis=1)
    h = x_ref[...] + agg
    h = jnp.dot(h, w1t_ref[...], preferred_element_type=jnp.float32)
    h = _bn(h + b1_ref[...], g1_ref[...], be1_ref[...])
    h = jnp.maximum(h, 0.0)
    h = jnp.dot(h, w2t_ref[...], preferred_element_type=jnp.float32)
    h = _bn(h + b2_ref[...], g2_ref[...], be2_ref[...])
    h = jnp.maximum(h, 0.0)
    out_ref[...] = _bn(h, g3_ref[...], be3_ref[...])


def kernel(x, edge_index, W1, b1, g1, be1, W2, b2, g2, be2, g3, be3):
    src = edge_index[0].reshape(NS, NCHUNK, CHUNK)
    dst = edge_index[1].reshape(NS, NCHUNK, CHUNK)
    xa = x[:, :DH]
    xb = x[:, DH:]
    parts = _sc_agg(xa, xb, src, dst)
    row = lambda v: v.reshape(1, -1)
    return pl.pallas_call(
        _mlp_kernel,
        out_shape=jax.ShapeDtypeStruct((N, D), jnp.float32),
    )(x, parts[0, :N], parts[1, :N], W1.T, row(b1), row(g1), row(be1),
      W2.T, row(b2), row(g2), row(be2), row(g3), row(be3))
